# Initial kernel scaffold; baseline (speedup 1.0000x reference)
import functools

import jax
import jax.numpy as jnp
from jax import lax
from jax.experimental import pallas as pl
from jax.experimental.pallas import tpu as pltpu
from jax.experimental.pallas import tpu_sc as plsc

N = 10000
E = 320000
DF = 128
DH = 128
DC = 64
NC = 2
NS = 16
NW = NC * NS
EPT = E // NW
CH = 80
NCHUNK = EPT // CH
RPT = N // NS
DEGW = 16

_MESH = plsc.VectorSubcoreMesh(core_axis_name="c", subcore_axis_name="s")


def _deg_body(dst_hbm, ones_hbm, z_hbm, out_hbm, dst_v, ones_v, acc_sh):
    c = lax.axis_index("c")
    s = lax.axis_index("s")
    w = c * NS + s
    r0 = s * RPT
    pltpu.sync_copy(z_hbm.at[pl.ds(r0, RPT)], acc_sh.at[pl.ds(r0, RPT)])
    pltpu.sync_copy(ones_hbm, ones_v)
    pltpu.sync_copy(dst_hbm.at[w], dst_v)
    plsc.subcore_barrier()

    @pl.loop(0, NCHUNK)
    def _(j):
        pltpu.sync_copy(ones_v, acc_sh.at[dst_v.at[j]], add=True)

    plsc.subcore_barrier()
    pltpu.sync_copy(acc_sh.at[pl.ds(r0, RPT)], out_hbm.at[c, pl.ds(r0, RPT)])


def _degree(dst, ones, zeros):
    return pl.kernel(
        _deg_body,
        out_type=jax.ShapeDtypeStruct((NC, N, DEGW), jnp.float32),
        mesh=_MESH,
        scratch_types=[
            pltpu.VMEM((NCHUNK, CH), jnp.int32),
            pltpu.VMEM((CH, DEGW), jnp.float32),
            pltpu.VMEM_SHARED((N, DEGW), jnp.float32),
        ],
    )(dst, ones, zeros)


def _prop_body(src_hbm, dst_hbm, h_hbm, z_hbm, out_hbm,
               src_v, dst_v, rows_v, acc_sh):
    c = lax.axis_index("c")
    s = lax.axis_index("s")
    w = c * NS + s
    r0 = s * RPT

    @pl.when(c == 0)
    def _():
        pltpu.sync_copy(h_hbm.at[pl.ds(r0, RPT)], acc_sh.at[pl.ds(r0, RPT)])

    @pl.when(c != 0)
    def _():
        pltpu.sync_copy(z_hbm.at[pl.ds(r0, RPT)], acc_sh.at[pl.ds(r0, RPT)])

    pltpu.sync_copy(src_hbm.at[w], src_v)
    pltpu.sync_copy(dst_hbm.at[w], dst_v)
    plsc.subcore_barrier()

    @pl.loop(0, NCHUNK)
    def _(j):
        pltpu.sync_copy(h_hbm.at[src_v.at[j]], rows_v)
        pltpu.sync_copy(rows_v, acc_sh.at[dst_v.at[j]], add=True)

    plsc.subcore_barrier()
    pltpu.sync_copy(acc_sh.at[pl.ds(r0, RPT)], out_hbm.at[c, pl.ds(r0, RPT)])


def _propagate(src, dst, h, zeros, d):
    return pl.kernel(
        _prop_body,
        out_type=jax.ShapeDtypeStruct((NC, N, d), jnp.float32),
        mesh=_MESH,
        scratch_types=[
            pltpu.VMEM((NCHUNK, CH), jnp.int32),
            pltpu.VMEM((NCHUNK, CH), jnp.int32),
            pltpu.VMEM((CH, d), jnp.float32),
            pltpu.VMEM_SHARED((N, d), jnp.float32),
        ],
    )(src, dst, h, zeros)


_BLK = 1000


def _dinv(deg_ref):
    return lax.rsqrt(1.0 + deg_ref[0, :, 0] + deg_ref[1, :, 0])


def _mm1_body(deg_ref, x_ref, w_ref, h_ref):
    dinv = _dinv(deg_ref)
    h = jnp.dot(x_ref[...], w_ref[...], preferred_element_type=jnp.float32,
                precision=lax.Precision.HIGHEST)
    h_ref[...] = h * dinv[:, None]


def _mm2_body(deg_ref, a_ref, b1_ref, w2_ref, h2_ref):
    dinv = _dinv(deg_ref)
    z = jnp.maximum((a_ref[0] + a_ref[1]) * dinv[:, None] + b1_ref[...], 0.0)
    h2 = jnp.dot(z, w2_ref[...], preferred_element_type=jnp.float32,
                 precision=lax.Precision.HIGHEST)
    h2_ref[...] = h2 * dinv[:, None]


def _fin_body(deg_ref, a_ref, b2_ref, o_ref):
    dinv = _dinv(deg_ref)
    o_ref[...] = (a_ref[0] + a_ref[1]) * dinv[:, None] + b2_ref[...]


def _deg_spec():
    return pl.BlockSpec((NC, _BLK, DEGW), lambda i: (0, i, 0))


def _mm1(deg, x, W1):
    return pl.pallas_call(
        _mm1_body,
        grid=(N // _BLK,),
        in_specs=[
            _deg_spec(),
            pl.BlockSpec((_BLK, DF), lambda i: (i, 0)),
            pl.BlockSpec((DF, DH), lambda i: (0, 0)),
        ],
        out_specs=pl.BlockSpec((_BLK, DH), lambda i: (i, 0)),
        out_shape=jax.ShapeDtypeStruct((N, DH), jnp.float32),
    )(deg, x, W1)


def _mm2(deg, acc1, b1, W2):
    return pl.pallas_call(
        _mm2_body,
        grid=(N // _BLK,),
        in_specs=[
            _deg_spec(),
            pl.BlockSpec((NC, _BLK, DH), lambda i: (0, i, 0)),
            pl.BlockSpec((1, DH), lambda i: (0, 0)),
            pl.BlockSpec((DH, DC), lambda i: (0, 0)),
        ],
        out_specs=pl.BlockSpec((_BLK, DC), lambda i: (i, 0)),
        out_shape=jax.ShapeDtypeStruct((N, DC), jnp.float32),
    )(deg, acc1, b1, W2)


def _fin(deg, acc2, b2):
    return pl.pallas_call(
        _fin_body,
        grid=(N // _BLK,),
        in_specs=[
            _deg_spec(),
            pl.BlockSpec((NC, _BLK, DC), lambda i: (0, i, 0)),
            pl.BlockSpec((1, DC), lambda i: (0, 0)),
        ],
        out_specs=pl.BlockSpec((_BLK, DC), lambda i: (i, 0)),
        out_shape=jax.ShapeDtypeStruct((N, DC), jnp.float32),
    )(deg, acc2, b2)


def kernel(x, edge_index, W1, b1, W2, b2):
    ei = edge_index.astype(jnp.int32)
    src = ei[0].reshape(NW, NCHUNK, CH)
    dst = ei[1].reshape(NW, NCHUNK, CH)
    ones = jnp.ones((CH, DEGW), jnp.float32)
    z16 = jnp.zeros((N, DEGW), jnp.float32)
    z128 = jnp.zeros((N, DH), jnp.float32)
    z64 = jnp.zeros((N, DC), jnp.float32)

    deg = _degree(dst, ones, z16)
    h1 = _mm1(deg, x, W1)
    acc1 = _propagate(src, dst, h1, z128, DH)
    h2 = _mm2(deg, acc1, b1.reshape(1, DH), W2)
    acc2 = _propagate(src, dst, h2, z64, DC)
    return _fin(deg, acc2, b2.reshape(1, DC))


# speedup vs baseline: 17.4912x; 17.4912x over previous
import functools

import jax
import jax.numpy as jnp
from jax import lax
from jax.experimental import pallas as pl
from jax.experimental.pallas import tpu as pltpu
from jax.experimental.pallas import tpu_sc as plsc

N = 10000
NP = 10240
E = 320000
DF = 128
DH = 128
DC = 64
NC = 2
NS = 16
NW = NC * NS
EPT = E // NW
CH = 80
NCHUNK = EPT // CH
RPT = NP // NS
DEGW = 128

_MESH = plsc.VectorSubcoreMesh(core_axis_name="c", subcore_axis_name="s")


def _deg_body(dst_hbm, ones_hbm, z_hbm, out_hbm, dst_v, ones_v, acc_sh):
    c = lax.axis_index("c")
    s = lax.axis_index("s")
    w = c * NS + s
    r0 = s * RPT
    pltpu.sync_copy(z_hbm.at[pl.ds(r0, RPT)], acc_sh.at[pl.ds(r0, RPT)])
    pltpu.sync_copy(ones_hbm, ones_v)
    pltpu.sync_copy(dst_hbm.at[w], dst_v)
    plsc.subcore_barrier()

    @pl.loop(0, NCHUNK)
    def _(j):
        pltpu.sync_copy(ones_v, acc_sh.at[dst_v.at[j]], add=True)

    plsc.subcore_barrier()
    pltpu.sync_copy(acc_sh.at[pl.ds(r0, RPT)], out_hbm.at[c, pl.ds(r0, RPT)])


def _degree(dst, ones, zeros):
    return pl.kernel(
        _deg_body,
        out_type=jax.ShapeDtypeStruct((NC, NP, DEGW), jnp.float32),
        mesh=_MESH,
        scratch_types=[
            pltpu.VMEM((NCHUNK, CH), jnp.int32),
            pltpu.VMEM((CH, DEGW), jnp.float32),
            pltpu.VMEM_SHARED((NP, DEGW), jnp.float32),
        ],
    )(dst, ones, zeros)


def _prop_body(src_hbm, dst_hbm, h_hbm, z_hbm, out_hbm,
               src_v, dst_v, rows_v, acc_sh):
    c = lax.axis_index("c")
    s = lax.axis_index("s")
    w = c * NS + s
    r0 = s * RPT

    @pl.when(c == 0)
    def _():
        pltpu.sync_copy(h_hbm.at[pl.ds(r0, RPT)], acc_sh.at[pl.ds(r0, RPT)])

    @pl.when(c != 0)
    def _():
        pltpu.sync_copy(z_hbm.at[pl.ds(r0, RPT)], acc_sh.at[pl.ds(r0, RPT)])

    pltpu.sync_copy(src_hbm.at[w], src_v)
    pltpu.sync_copy(dst_hbm.at[w], dst_v)
    plsc.subcore_barrier()

    @pl.loop(0, NCHUNK)
    def _(j):
        pltpu.sync_copy(h_hbm.at[src_v.at[j]], rows_v)
        pltpu.sync_copy(rows_v, acc_sh.at[dst_v.at[j]], add=True)

    plsc.subcore_barrier()
    pltpu.sync_copy(acc_sh.at[pl.ds(r0, RPT)], out_hbm.at[c, pl.ds(r0, RPT)])


def _propagate(src, dst, h, zeros, d):
    return pl.kernel(
        _prop_body,
        out_type=jax.ShapeDtypeStruct((NC, NP, d), jnp.float32),
        mesh=_MESH,
        scratch_types=[
            pltpu.VMEM((NCHUNK, CH), jnp.int32),
            pltpu.VMEM((NCHUNK, CH), jnp.int32),
            pltpu.VMEM((CH, d), jnp.float32),
            pltpu.VMEM_SHARED((NP, d), jnp.float32),
        ],
    )(src, dst, h, zeros)


_BLK = 1024


def _dinv(deg_ref):
    return lax.rsqrt(1.0 + deg_ref[0, :, 0] + deg_ref[1, :, 0])


def _mm1_body(deg_ref, x_ref, w_ref, h_ref):
    dinv = _dinv(deg_ref)
    h = jnp.dot(x_ref[...], w_ref[...], preferred_element_type=jnp.float32,
                precision=lax.Precision.HIGHEST)
    h_ref[...] = h * dinv[:, None]


def _mm2_body(deg_ref, a_ref, b1_ref, w2_ref, h2_ref):
    dinv = _dinv(deg_ref)
    z = jnp.maximum((a_ref[0] + a_ref[1]) * dinv[:, None] + b1_ref[...], 0.0)
    h2 = jnp.dot(z, w2_ref[...], preferred_element_type=jnp.float32,
                 precision=lax.Precision.HIGHEST)
    h2_ref[...] = h2 * dinv[:, None]


def _fin_body(deg_ref, a_ref, b2_ref, o_ref):
    dinv = _dinv(deg_ref)
    o_ref[...] = (a_ref[0, :, :DC] + a_ref[1, :, :DC]) * dinv[:, None] + b2_ref[...]


def _deg_spec():
    return pl.BlockSpec((NC, _BLK, DEGW), lambda i: (0, i, 0))


def _mm1(deg, x, W1):
    return pl.pallas_call(
        _mm1_body,
        grid=(NP // _BLK,),
        in_specs=[
            _deg_spec(),
            pl.BlockSpec((_BLK, DF), lambda i: (i, 0)),
            pl.BlockSpec((DF, DH), lambda i: (0, 0)),
        ],
        out_specs=pl.BlockSpec((_BLK, DH), lambda i: (i, 0)),
        out_shape=jax.ShapeDtypeStruct((NP, DH), jnp.float32),
    )(deg, x, W1)


def _mm2(deg, acc1, b1, W2):
    return pl.pallas_call(
        _mm2_body,
        grid=(NP // _BLK,),
        in_specs=[
            _deg_spec(),
            pl.BlockSpec((NC, _BLK, DH), lambda i: (0, i, 0)),
            pl.BlockSpec((1, DH), lambda i: (0, 0)),
            pl.BlockSpec((DH, DH), lambda i: (0, 0)),
        ],
        out_specs=pl.BlockSpec((_BLK, DH), lambda i: (i, 0)),
        out_shape=jax.ShapeDtypeStruct((NP, DH), jnp.float32),
    )(deg, acc1, b1, W2)


def _fin(deg, acc2, b2):
    return pl.pallas_call(
        _fin_body,
        grid=(NP // _BLK,),
        in_specs=[
            _deg_spec(),
            pl.BlockSpec((NC, _BLK, DH), lambda i: (0, i, 0)),
            pl.BlockSpec((1, DC), lambda i: (0, 0)),
        ],
        out_specs=pl.BlockSpec((_BLK, DC), lambda i: (i, 0)),
        out_shape=jax.ShapeDtypeStruct((NP, DC), jnp.float32),
    )(deg, acc2, b2)


def kernel(x, edge_index, W1, b1, W2, b2):
    ei = edge_index.astype(jnp.int32)
    src = ei[0].reshape(NW, NCHUNK, CH)
    dst = ei[1].reshape(NW, NCHUNK, CH)
    ones = jnp.ones((CH, DEGW), jnp.float32)
    z128 = jnp.zeros((NP, DH), jnp.float32)
    xp = jnp.pad(x, ((0, NP - N), (0, 0)))
    W2p = jnp.pad(W2, ((0, 0), (0, DH - DC)))

    deg = _degree(dst, ones, z128)
    h1 = _mm1(deg, xp, W1)
    acc1 = _propagate(src, dst, h1, z128, DH)
    h2 = _mm2(deg, acc1, b1.reshape(1, DH), W2p)
    acc2 = _propagate(src, dst, h2, z128, DH)
    return _fin(deg, acc2, b2.reshape(1, DC))[:N]


# speedup vs baseline: 24.1913x; 1.3831x over previous
import functools

import jax
import jax.numpy as jnp
from jax import lax
from jax.experimental import pallas as pl
from jax.experimental.pallas import tpu as pltpu
from jax.experimental.pallas import tpu_sc as plsc

N = 10000
NP = 10240
E = 320000
DF = 128
DH = 128
DC = 64
NC = 2
NS = 16
NW = NC * NS
EPT = E // NW
CH = 80
NCHUNK = EPT // CH
SEG = 25
NSEG = NCHUNK // SEG
RPT = NP // NS
DEGW = 128

_MESH = plsc.VectorSubcoreMesh(core_axis_name="c", subcore_axis_name="s")


def _deg_body(dst_hbm, ones_hbm, z_hbm, out_hbm, dst_v, ones_v, acc_sh, sem):
    c = lax.axis_index("c")
    s = lax.axis_index("s")
    w = c * NS + s
    r0 = s * RPT
    pltpu.sync_copy(z_hbm.at[pl.ds(r0, RPT)], acc_sh.at[pl.ds(r0, RPT)])
    pltpu.sync_copy(ones_hbm, ones_v)
    pltpu.sync_copy(dst_hbm.at[w], dst_v)
    plsc.subcore_barrier()

    @pl.loop(0, NCHUNK)
    def _(j):
        pltpu.sync_copy(ones_v, acc_sh.at[dst_v.at[j]], add=True)

    plsc.subcore_barrier()
    pltpu.sync_copy(acc_sh.at[pl.ds(r0, RPT)], out_hbm.at[c, pl.ds(r0, RPT)])


def _degree(dst, ones, zeros):
    return pl.kernel(
        _deg_body,
        out_type=jax.ShapeDtypeStruct((NC, NP, DEGW), jnp.float32),
        mesh=_MESH,
        scratch_types=[
            pltpu.VMEM((NCHUNK, CH), jnp.int32),
            pltpu.VMEM((CH, DEGW), jnp.float32),
            pltpu.VMEM_SHARED((NP, DEGW), jnp.float32),
            pltpu.SemaphoreType.DMA,
        ],
    )(dst, ones, zeros)


def _prop_body(src_hbm, dst_hbm, h_hbm, z_hbm, out_hbm,
               src_v, dst_v, rows0_v, rows1_v, acc_sh, sem0, sem1):
    c = lax.axis_index("c")
    s = lax.axis_index("s")
    w = c * NS + s
    r0 = s * RPT

    @pl.when(c == 0)
    def _():
        pltpu.sync_copy(h_hbm.at[pl.ds(r0, RPT)], acc_sh.at[pl.ds(r0, RPT)])

    @pl.when(c != 0)
    def _():
        pltpu.sync_copy(z_hbm.at[pl.ds(r0, RPT)], acc_sh.at[pl.ds(r0, RPT)])

    plsc.subcore_barrier()

    def gat(j, buf, sem):
        return pltpu.make_async_copy(h_hbm.at[src_v.at[j]], buf, sem)

    @pl.loop(0, NSEG)
    def _(g):
        pltpu.sync_copy(src_hbm.at[w, g], src_v)
        pltpu.sync_copy(dst_hbm.at[w, g], dst_v)
        gat(0, rows0_v, sem0).start()

        @pl.loop(0, SEG - 1, step=2)
        def _(j):
            gat(j + 1, rows1_v, sem1).start()
            gat(j, rows0_v, sem0).wait()
            pltpu.sync_copy(rows0_v, acc_sh.at[dst_v.at[j]], add=True)
            gat(j + 2, rows0_v, sem0).start()
            gat(j + 1, rows1_v, sem1).wait()
            pltpu.sync_copy(rows1_v, acc_sh.at[dst_v.at[j + 1]], add=True)

        gat(SEG - 1, rows0_v, sem0).wait()
        pltpu.sync_copy(rows0_v, acc_sh.at[dst_v.at[SEG - 1]], add=True)

    plsc.subcore_barrier()
    pltpu.sync_copy(acc_sh.at[pl.ds(r0, RPT)], out_hbm.at[c, pl.ds(r0, RPT)])


def _propagate(src, dst, h, zeros, d):
    return pl.kernel(
        _prop_body,
        out_type=jax.ShapeDtypeStruct((NC, NP, d), jnp.float32),
        mesh=_MESH,
        scratch_types=[
            pltpu.VMEM((SEG, CH), jnp.int32),
            pltpu.VMEM((SEG, CH), jnp.int32),
            pltpu.VMEM((CH, d), jnp.float32),
            pltpu.VMEM((CH, d), jnp.float32),
            pltpu.VMEM_SHARED((NP, d), jnp.float32),
            pltpu.SemaphoreType.DMA,
            pltpu.SemaphoreType.DMA,
        ],
    )(src, dst, h, zeros)


_BLK = 1024


def _dinv(deg_ref):
    return lax.rsqrt(1.0 + deg_ref[0, :, 0] + deg_ref[1, :, 0])


def _mm1_body(deg_ref, x_ref, w_ref, h_ref):
    dinv = _dinv(deg_ref)
    h = jnp.dot(x_ref[...], w_ref[...], preferred_element_type=jnp.float32,
                precision=lax.Precision.HIGHEST)
    h_ref[...] = h * dinv[:, None]


def _mm2_body(deg_ref, a_ref, b1_ref, w2_ref, h2_ref):
    dinv = _dinv(deg_ref)
    z = jnp.maximum((a_ref[0] + a_ref[1]) * dinv[:, None] + b1_ref[...], 0.0)
    h2 = jnp.dot(z, w2_ref[...], preferred_element_type=jnp.float32,
                 precision=lax.Precision.HIGHEST)
    h2_ref[...] = h2 * dinv[:, None]


def _fin_body(deg_ref, a_ref, b2_ref, o_ref):
    dinv = _dinv(deg_ref)
    o_ref[...] = (a_ref[0, :, :DC] + a_ref[1, :, :DC]) * dinv[:, None] + b2_ref[...]


def _deg_spec():
    return pl.BlockSpec((NC, _BLK, DEGW), lambda i: (0, i, 0))


def _mm1(deg, x, W1):
    return pl.pallas_call(
        _mm1_body,
        grid=(NP // _BLK,),
        in_specs=[
            _deg_spec(),
            pl.BlockSpec((_BLK, DF), lambda i: (i, 0)),
            pl.BlockSpec((DF, DH), lambda i: (0, 0)),
        ],
        out_specs=pl.BlockSpec((_BLK, DH), lambda i: (i, 0)),
        out_shape=jax.ShapeDtypeStruct((NP, DH), jnp.float32),
    )(deg, x, W1)


def _mm2(deg, acc1, b1, W2):
    return pl.pallas_call(
        _mm2_body,
        grid=(NP // _BLK,),
        in_specs=[
            _deg_spec(),
            pl.BlockSpec((NC, _BLK, DH), lambda i: (0, i, 0)),
            pl.BlockSpec((1, DH), lambda i: (0, 0)),
            pl.BlockSpec((DH, DH), lambda i: (0, 0)),
        ],
        out_specs=pl.BlockSpec((_BLK, DH), lambda i: (i, 0)),
        out_shape=jax.ShapeDtypeStruct((NP, DH), jnp.float32),
    )(deg, acc1, b1, W2)


def _fin(deg, acc2, b2):
    return pl.pallas_call(
        _fin_body,
        grid=(NP // _BLK,),
        in_specs=[
            _deg_spec(),
            pl.BlockSpec((NC, _BLK, DH), lambda i: (0, i, 0)),
            pl.BlockSpec((1, DC), lambda i: (0, 0)),
        ],
        out_specs=pl.BlockSpec((_BLK, DC), lambda i: (i, 0)),
        out_shape=jax.ShapeDtypeStruct((NP, DC), jnp.float32),
    )(deg, acc2, b2)


def kernel(x, edge_index, W1, b1, W2, b2):
    ei = edge_index.astype(jnp.int32)
    src = ei[0].reshape(NW, NCHUNK, CH)
    dst = ei[1].reshape(NW, NCHUNK, CH)
    src4 = ei[0].reshape(NW, NSEG, SEG, CH)
    dst4 = ei[1].reshape(NW, NSEG, SEG, CH)
    ones = jnp.ones((CH, DEGW), jnp.float32)
    z128 = jnp.zeros((NP, DH), jnp.float32)
    xp = jnp.pad(x, ((0, NP - N), (0, 0)))
    W2p = jnp.pad(W2, ((0, 0), (0, DH - DC)))

    deg = _degree(dst, ones, z128)
    h1 = _mm1(deg, xp, W1)
    acc1 = _propagate(src4, dst4, h1, z128, DH)
    h2 = _mm2(deg, acc1, b1.reshape(1, DH), W2p)
    acc2 = _propagate(src4, dst4, h2, z128, DH)
    return _fin(deg, acc2, b2.reshape(1, DC))[:N]


# speedup vs baseline: 25.3067x; 1.0461x over previous
import functools

import jax
import jax.numpy as jnp
from jax import lax
from jax.experimental import pallas as pl
from jax.experimental.pallas import tpu as pltpu
from jax.experimental.pallas import tpu_sc as plsc

N = 10000
NP = 10240
E = 320000
DF = 128
DH = 128
DC = 64
NC = 2
NS = 16
NW = NC * NS
EPT = E // NW
CH = 100
NCHUNK = EPT // CH
SEG = 25
NSEG = NCHUNK // SEG
RPT = NP // NS
DEGW = 128

_MESH = plsc.VectorSubcoreMesh(core_axis_name="c", subcore_axis_name="s")


def _deg_body(dst_hbm, ones_hbm, z_hbm, out_hbm, dst_v, ones_v, acc_sh, sem):
    c = lax.axis_index("c")
    s = lax.axis_index("s")
    w = c * NS + s
    r0 = s * RPT
    pltpu.sync_copy(z_hbm.at[pl.ds(r0, RPT)], acc_sh.at[pl.ds(r0, RPT)])
    pltpu.sync_copy(ones_hbm, ones_v)
    pltpu.sync_copy(dst_hbm.at[w], dst_v)
    plsc.subcore_barrier()

    @pl.loop(0, NCHUNK)
    def _(j):
        pltpu.sync_copy(ones_v, acc_sh.at[dst_v.at[j]], add=True)

    plsc.subcore_barrier()
    pltpu.sync_copy(acc_sh.at[pl.ds(r0, RPT)], out_hbm.at[c, pl.ds(r0, RPT)])


def _degree(dst, ones, zeros):
    return pl.kernel(
        _deg_body,
        out_type=jax.ShapeDtypeStruct((NC, NP, DEGW), jnp.float32),
        mesh=_MESH,
        scratch_types=[
            pltpu.VMEM((NCHUNK, CH), jnp.int32),
            pltpu.VMEM((CH, DEGW), jnp.float32),
            pltpu.VMEM_SHARED((NP, DEGW), jnp.float32),
            pltpu.SemaphoreType.DMA,
        ],
    )(dst, ones, zeros)


def _prop_body(src_hbm, dst_hbm, h_hbm, z_hbm, out_hbm,
               src_v, dst_v, rows0_v, rows1_v, acc_sh, sem0, sem1):
    c = lax.axis_index("c")
    s = lax.axis_index("s")
    w = c * NS + s
    r0 = s * RPT

    @pl.when(c == 0)
    def _():
        pltpu.sync_copy(h_hbm.at[pl.ds(r0, RPT)], acc_sh.at[pl.ds(r0, RPT)])

    @pl.when(c != 0)
    def _():
        pltpu.sync_copy(z_hbm.at[pl.ds(r0, RPT)], acc_sh.at[pl.ds(r0, RPT)])

    plsc.subcore_barrier()

    def gat(j, buf, sem):
        return pltpu.make_async_copy(h_hbm.at[src_v.at[j]], buf, sem)

    @pl.loop(0, NSEG)
    def _(g):
        pltpu.sync_copy(src_hbm.at[w, g], src_v)
        pltpu.sync_copy(dst_hbm.at[w, g], dst_v)
        gat(0, rows0_v, sem0).start()

        @pl.loop(0, SEG - 1, step=2)
        def _(j):
            gat(j + 1, rows1_v, sem1).start()
            gat(j, rows0_v, sem0).wait()
            pltpu.sync_copy(rows0_v, acc_sh.at[dst_v.at[j]], add=True)
            gat(j + 2, rows0_v, sem0).start()
            gat(j + 1, rows1_v, sem1).wait()
            pltpu.sync_copy(rows1_v, acc_sh.at[dst_v.at[j + 1]], add=True)

        gat(SEG - 1, rows0_v, sem0).wait()
        pltpu.sync_copy(rows0_v, acc_sh.at[dst_v.at[SEG - 1]], add=True)

    plsc.subcore_barrier()
    pltpu.sync_copy(acc_sh.at[pl.ds(r0, RPT)], out_hbm.at[c, pl.ds(r0, RPT)])


def _propagate(src, dst, h, zeros, d):
    return pl.kernel(
        _prop_body,
        out_type=jax.ShapeDtypeStruct((NC, NP, d), jnp.float32),
        mesh=_MESH,
        scratch_types=[
            pltpu.VMEM((SEG, CH), jnp.int32),
            pltpu.VMEM((SEG, CH), jnp.int32),
            pltpu.VMEM((CH, d), jnp.float32),
            pltpu.VMEM((CH, d), jnp.float32),
            pltpu.VMEM_SHARED((NP, d), jnp.float32),
            pltpu.SemaphoreType.DMA,
            pltpu.SemaphoreType.DMA,
        ],
    )(src, dst, h, zeros)


_BLK = 1024


def _dinv(deg_ref):
    return lax.rsqrt(1.0 + deg_ref[0, :, 0] + deg_ref[1, :, 0])


def _mm1_body(deg_ref, x_ref, w_ref, h_ref):
    dinv = _dinv(deg_ref)
    h = jnp.dot(x_ref[...], w_ref[...], preferred_element_type=jnp.float32,
                precision=lax.Precision.HIGHEST)
    h_ref[...] = h * dinv[:, None]


def _mm2_body(deg_ref, a_ref, b1_ref, w2_ref, h2_ref):
    dinv = _dinv(deg_ref)
    z = jnp.maximum((a_ref[0] + a_ref[1]) * dinv[:, None] + b1_ref[...], 0.0)
    h2 = jnp.dot(z, w2_ref[...], preferred_element_type=jnp.float32,
                 precision=lax.Precision.HIGHEST)
    h2_ref[...] = h2 * dinv[:, None]


def _fin_body(deg_ref, a_ref, b2_ref, o_ref):
    dinv = _dinv(deg_ref)
    o_ref[...] = (a_ref[0, :, :DC] + a_ref[1, :, :DC]) * dinv[:, None] + b2_ref[...]


def _deg_spec():
    return pl.BlockSpec((NC, _BLK, DEGW), lambda i: (0, i, 0))


def _mm1(deg, x, W1):
    return pl.pallas_call(
        _mm1_body,
        grid=(NP // _BLK,),
        in_specs=[
            _deg_spec(),
            pl.BlockSpec((_BLK, DF), lambda i: (i, 0)),
            pl.BlockSpec((DF, DH), lambda i: (0, 0)),
        ],
        out_specs=pl.BlockSpec((_BLK, DH), lambda i: (i, 0)),
        out_shape=jax.ShapeDtypeStruct((NP, DH), jnp.float32),
    )(deg, x, W1)


def _mm2(deg, acc1, b1, W2):
    return pl.pallas_call(
        _mm2_body,
        grid=(NP // _BLK,),
        in_specs=[
            _deg_spec(),
            pl.BlockSpec((NC, _BLK, DH), lambda i: (0, i, 0)),
            pl.BlockSpec((1, DH), lambda i: (0, 0)),
            pl.BlockSpec((DH, DH), lambda i: (0, 0)),
        ],
        out_specs=pl.BlockSpec((_BLK, DH), lambda i: (i, 0)),
        out_shape=jax.ShapeDtypeStruct((NP, DH), jnp.float32),
    )(deg, acc1, b1, W2)


def _fin(deg, acc2, b2):
    return pl.pallas_call(
        _fin_body,
        grid=(NP // _BLK,),
        in_specs=[
            _deg_spec(),
            pl.BlockSpec((NC, _BLK, DH), lambda i: (0, i, 0)),
            pl.BlockSpec((1, DC), lambda i: (0, 0)),
        ],
        out_specs=pl.BlockSpec((_BLK, DC), lambda i: (i, 0)),
        out_shape=jax.ShapeDtypeStruct((NP, DC), jnp.float32),
    )(deg, acc2, b2)


def kernel(x, edge_index, W1, b1, W2, b2):
    ei = edge_index.astype(jnp.int32)
    src = ei[0].reshape(NW, NCHUNK, CH)
    dst = ei[1].reshape(NW, NCHUNK, CH)
    src4 = ei[0].reshape(NW, NSEG, SEG, CH)
    dst4 = ei[1].reshape(NW, NSEG, SEG, CH)
    ones = jnp.ones((CH, DEGW), jnp.float32)
    z128 = jnp.zeros((NP, DH), jnp.float32)
    xp = jnp.pad(x, ((0, NP - N), (0, 0)))
    W2p = jnp.pad(W2, ((0, 0), (0, DH - DC)))

    deg = _degree(dst, ones, z128)
    h1 = _mm1(deg, xp, W1)
    acc1 = _propagate(src4, dst4, h1, z128, DH)
    h2 = _mm2(deg, acc1, b1.reshape(1, DH), W2p)
    acc2 = _propagate(src4, dst4, h2, z128, DH)
    return _fin(deg, acc2, b2.reshape(1, DC))[:N]
